# baseline (speedup 1.0000x reference)
import functools

import jax
import jax.numpy as jnp
from jax import lax
from jax.experimental import pallas as pl
from jax.experimental.pallas import tpu as pltpu
from jax.experimental.pallas import tpu_sc as plsc

_SELECTION_RATIO = 0.7
_N_BISECT = 30
_SC_ROWS = 3584
_NW = 32
_BLK = 512


def _dot(a, b, precision=None):
    return jax.lax.dot_general(
        a, b, (((1,), (0,)), ((), ())),
        preferred_element_type=jnp.float32,
        precision=precision,
    )


def _mlp_body(x_ref, w1_ref, b1_ref, w2_ref, b2_ref, wg1_ref, bg1_ref,
              wg2_ref, bg2_ref, comb_ref, bits_ref):
    x = x_ref[...]
    h = jnp.maximum(_dot(x, w1_ref[...]) + b1_ref[...], 0.0)
    att = jax.nn.sigmoid(_dot(h, w2_ref[...]) + b2_ref[...])
    hg = jnp.maximum(_dot(x, wg1_ref[...]) + bg1_ref[...], 0.0)
    gate = jax.nn.sigmoid(_dot(hg, wg2_ref[...]) + bg2_ref[...])
    s = att * gate
    comb_ref[...] = s
    bits_ref[...] = jax.lax.bitcast_convert_type(s, jnp.int32)


def _select_body(k, x_ref, comb_ref, sel_ref, mask_ref):
    x = x_ref[...]
    s = comb_ref[...]
    ki = jax.lax.bitcast_convert_type(s, jnp.int32)
    rows = s.shape[0]
    kit = ki.T
    lo_t = jnp.zeros((1, rows), jnp.int32)
    hi_t = jnp.full((1, rows), 0x40000000, jnp.int32)
    for _ in range(_N_BISECT):
        mid = (lo_t + hi_t) >> 1
        cnt = jnp.sum((kit >= mid).astype(jnp.int32), axis=0, keepdims=True)
        big = cnt >= k
        lo_t = jnp.where(big, mid, lo_t)
        hi_t = jnp.where(big, hi_t, mid)
    cgt_t = jnp.sum((kit > lo_t).astype(jnp.int32), axis=0, keepdims=True)
    thr = lo_t.T
    c_gt = cgt_t.T

    gt = ki > thr
    eq = ki == thr
    eq_f = eq.astype(jnp.float32)
    d = s.shape[1]
    tri = (jax.lax.broadcasted_iota(jnp.int32, (d, d), 0)
           < jax.lax.broadcasted_iota(jnp.int32, (d, d), 1)).astype(jnp.float32)
    prefix = _dot(eq_f, tri)
    sel_eq = eq & (prefix < (k - c_gt).astype(jnp.float32))
    mask = (gt | sel_eq).astype(jnp.float32)

    mask_ref[...] = mask
    sel_ref[...] = x * mask


def _sc_mask_body(k, rows_w, x_hbm, comb_hbm, mask_hbm, sel_hbm,
                  xv, cv, mv, sv):
    wid = lax.axis_index("s") * 2 + lax.axis_index("c")
    base = wid * rows_w
    pltpu.sync_copy(comb_hbm.at[pl.ds(base, rows_w)], cv)
    pltpu.sync_copy(x_hbm.at[pl.ds(base, rows_w)], xv)

    kvec = jnp.full((16,), k, jnp.int32)
    one_f = jnp.full((16,), 1.0, jnp.float32)
    zero_f = jnp.zeros((16,), jnp.float32)

    def row_body(r, carry):
        kis = [cv[r, pl.ds(16 * v, 16)] for v in range(8)]
        lo = jnp.zeros((16,), jnp.int32)
        hi = jnp.full((16,), 0x40000000, jnp.int32)
        for _ in range(_N_BISECT):
            mid = (lo + hi) >> 1
            cnt = jnp.zeros((16,), jnp.int32)
            for v in range(8):
                cnt = cnt + plsc.all_reduce_population_count(kis[v] >= mid)
            big = cnt >= kvec
            lo = jnp.where(big, mid, lo)
            hi = jnp.where(big, hi, mid)
        thr = lo
        c_gt = jnp.zeros((16,), jnp.int32)
        for v in range(8):
            c_gt = c_gt + plsc.all_reduce_population_count(kis[v] > thr)
        limit = kvec - c_gt
        run = jnp.zeros((16,), jnp.int32)
        for v in range(8):
            gt = kis[v] > thr
            eq = kis[v] == thr
            eq_i = jnp.where(eq, 1, 0)
            excl = plsc.cumsum(eq_i) - eq_i + run
            sel_eq = eq & (excl < limit)
            m = jnp.where(gt | sel_eq, one_f, zero_f)
            mv[r, pl.ds(16 * v, 16)] = m
            sv[r, pl.ds(16 * v, 16)] = xv[r, pl.ds(16 * v, 16)] * m
            run = run + plsc.all_reduce_population_count(eq)
        return carry

    lax.fori_loop(0, rows_w, row_body, 0)
    pltpu.sync_copy(mv, mask_hbm.at[pl.ds(base, rows_w)])
    pltpu.sync_copy(sv, sel_hbm.at[pl.ds(base, rows_w)])


def _sc_mask(x, comb_bits, sc_rows, k):
    d = x.shape[1]
    rows_w = sc_rows // _NW
    mesh = plsc.VectorSubcoreMesh(core_axis_name="c", subcore_axis_name="s")
    fn = pl.kernel(
        functools.partial(_sc_mask_body, k, rows_w),
        out_type=[jax.ShapeDtypeStruct((sc_rows, d), jnp.float32)] * 2,
        mesh=mesh,
        compiler_params=pltpu.CompilerParams(
            needs_layout_passes=False, vmem_limit_bytes=1024 * 1024),
        cost_estimate=pl.CostEstimate(
            flops=sc_rows * d * 40,
            transcendentals=0,
            bytes_accessed=sc_rows * d * 16,
        ),
        scratch_types=[
            pltpu.VMEM((rows_w, d), jnp.float32),
            pltpu.VMEM((rows_w, d), jnp.int32),
            pltpu.VMEM((rows_w, d), jnp.float32),
            pltpu.VMEM((rows_w, d), jnp.float32),
        ],
    )
    mask_sc, sel_sc = fn(x, comb_bits)
    return mask_sc, sel_sc


def kernel(x, W1, b1, W2, b2, Wg1, bg1, Wg2, bg2):
    bsz, d = x.shape
    hdim = W1.shape[1]
    k = int(_SELECTION_RATIO * d)
    blk = min(bsz, _BLK)
    n_blocks = bsz // blk
    sc_rows = _SC_ROWS if bsz % blk == 0 and _SC_ROWS % blk == 0 \
        and _SC_ROWS % _NW == 0 and bsz - _SC_ROWS >= blk else 0
    n_sc_blocks = sc_rows // blk

    b1r = b1.reshape(1, hdim)
    b2r = b2.reshape(1, d)
    bg1r = bg1.reshape(1, hdim)
    bg2r = bg2.reshape(1, 1)

    full = lambda shape: pl.BlockSpec(shape, lambda i: (0, 0))
    rowblk = lambda shape: pl.BlockSpec(shape, lambda i: (i, 0))

    comb, comb_bits = pl.pallas_call(
        _mlp_body,
        grid=(n_blocks,),
        in_specs=[
            rowblk((blk, d)),
            full((d, hdim)), full((1, hdim)),
            full((hdim, d)), full((1, d)),
            full((d, hdim)), full((1, hdim)),
            full((hdim, 1)), full((1, 1)),
        ],
        out_specs=[rowblk((blk, d))] * 2,
        out_shape=[jax.ShapeDtypeStruct((bsz, d), jnp.float32),
                   jax.ShapeDtypeStruct((bsz, d), jnp.int32)],
        compiler_params=pltpu.CompilerParams(
            vmem_limit_bytes=24 * 1024 * 1024),
    )(x, W1, b1r, W2, b2r, Wg1, bg1r, Wg2, bg2r)

    if sc_rows:
        mask_sc, sel_sc = _sc_mask(x, comb_bits, sc_rows, k)

    off = lambda shape: pl.BlockSpec(shape, lambda i: (i + n_sc_blocks, 0))
    sel_tc, mask_tc = pl.pallas_call(
        functools.partial(_select_body, k),
        grid=(n_blocks - n_sc_blocks,),
        in_specs=[off((blk, d)), off((blk, d))],
        out_specs=[off((blk, d))] * 2,
        out_shape=[jax.ShapeDtypeStruct((bsz, d), jnp.float32)] * 2,
        compiler_params=pltpu.CompilerParams(
            vmem_limit_bytes=24 * 1024 * 1024),
    )(x, comb)

    if sc_rows == 0:
        return (sel_tc, comb, mask_tc)
    mask = lax.dynamic_update_slice(mask_tc, mask_sc, (0, 0))
    sel = lax.dynamic_update_slice(sel_tc, sel_sc, (0, 0))
    return (sel, comb, mask)


# speedup vs baseline: 1.0844x; 1.0844x over previous
import functools

import jax
import jax.numpy as jnp
from jax import lax
from jax.experimental import pallas as pl
from jax.experimental.pallas import tpu as pltpu
from jax.experimental.pallas import tpu_sc as plsc

_SELECTION_RATIO = 0.7
_N_BISECT = 30
_SC_ROWS = 2048
_NW = 32
_BLK = 1024


def _dot(a, b, precision=None):
    return jax.lax.dot_general(
        a, b, (((1,), (0,)), ((), ())),
        preferred_element_type=jnp.float32,
        precision=precision,
    )


def _tc_body(k, n_sc_blocks, x_ref, w1_ref, b1_ref, w2_ref, b2_ref, wg1_ref,
             bg1_ref, wg2_ref, bg2_ref, sel_ref, comb_ref, bits_ref,
             mask_ref):
    x = x_ref[...]
    h = jnp.maximum(_dot(x, w1_ref[...]) + b1_ref[...], 0.0)
    att = jax.nn.sigmoid(_dot(h, w2_ref[...]) + b2_ref[...])
    hg = jnp.maximum(_dot(x, wg1_ref[...]) + bg1_ref[...], 0.0)
    gate = jax.nn.sigmoid(_dot(hg, wg2_ref[...]) + bg2_ref[...])
    s = att * gate
    comb_ref[...] = s
    ki = jax.lax.bitcast_convert_type(s, jnp.int32)
    bits_ref[...] = ki

    @pl.when(pl.program_id(0) >= n_sc_blocks)
    def _selection():
        rows = s.shape[0]
        kit = ki.T
        lo_t = jnp.zeros((1, rows), jnp.int32)
        hi_t = jnp.full((1, rows), 0x40000000, jnp.int32)
        for _ in range(_N_BISECT):
            mid = (lo_t + hi_t) >> 1
            cnt = jnp.sum((kit >= mid).astype(jnp.int32), axis=0,
                          keepdims=True)
            big = cnt >= k
            lo_t = jnp.where(big, mid, lo_t)
            hi_t = jnp.where(big, hi_t, mid)
        cgt_t = jnp.sum((kit > lo_t).astype(jnp.int32), axis=0,
                        keepdims=True)
        thr = lo_t.T
        c_gt = cgt_t.T

        gt = ki > thr
        eq = ki == thr
        eq_f = eq.astype(jnp.float32)
        d = s.shape[1]
        tri = (jax.lax.broadcasted_iota(jnp.int32, (d, d), 0)
               < jax.lax.broadcasted_iota(jnp.int32, (d, d), 1)
               ).astype(jnp.float32)
        prefix = _dot(eq_f, tri)
        sel_eq = eq & (prefix < (k - c_gt).astype(jnp.float32))
        mask = (gt | sel_eq).astype(jnp.float32)

        mask_ref[...] = mask
        sel_ref[...] = x * mask


def _sc_mask_body(k, rows_w, x_hbm, bits_hbm, mask_hbm, sel_hbm,
                  xv, cv, mv, sv):
    wid = lax.axis_index("s") * 2 + lax.axis_index("c")
    base = wid * rows_w
    pltpu.sync_copy(bits_hbm.at[pl.ds(base, rows_w)], cv)
    pltpu.sync_copy(x_hbm.at[pl.ds(base, rows_w)], xv)

    kvec = jnp.full((16,), k, jnp.int32)
    one_f = jnp.full((16,), 1.0, jnp.float32)
    zero_f = jnp.zeros((16,), jnp.float32)

    def row_body(r, carry):
        kis = [cv[r, pl.ds(16 * v, 16)] for v in range(8)]
        lo = jnp.zeros((16,), jnp.int32)
        hi = jnp.full((16,), 0x40000000, jnp.int32)
        for _ in range(_N_BISECT):
            mid = (lo + hi) >> 1
            cnt = jnp.zeros((16,), jnp.int32)
            for v in range(8):
                cnt = cnt + plsc.all_reduce_population_count(kis[v] >= mid)
            big = cnt >= kvec
            lo = jnp.where(big, mid, lo)
            hi = jnp.where(big, hi, mid)
        thr = lo
        c_gt = jnp.zeros((16,), jnp.int32)
        for v in range(8):
            c_gt = c_gt + plsc.all_reduce_population_count(kis[v] > thr)
        limit = kvec - c_gt
        run = jnp.zeros((16,), jnp.int32)
        for v in range(8):
            gt = kis[v] > thr
            eq = kis[v] == thr
            eq_i = jnp.where(eq, 1, 0)
            excl = plsc.cumsum(eq_i) - eq_i + run
            sel_eq = eq & (excl < limit)
            m = jnp.where(gt | sel_eq, one_f, zero_f)
            mv[r, pl.ds(16 * v, 16)] = m
            sv[r, pl.ds(16 * v, 16)] = xv[r, pl.ds(16 * v, 16)] * m
            run = run + plsc.all_reduce_population_count(eq)
        return carry

    lax.fori_loop(0, rows_w, row_body, 0)
    pltpu.sync_copy(mv, mask_hbm.at[pl.ds(base, rows_w)])
    pltpu.sync_copy(sv, sel_hbm.at[pl.ds(base, rows_w)])


def _sc_mask(x, bits, sc_rows, k):
    d = x.shape[1]
    rows_w = sc_rows // _NW
    mesh = plsc.VectorSubcoreMesh(core_axis_name="c", subcore_axis_name="s")
    fn = pl.kernel(
        functools.partial(_sc_mask_body, k, rows_w),
        out_type=[jax.ShapeDtypeStruct((sc_rows, d), jnp.float32)] * 2,
        mesh=mesh,
        compiler_params=pltpu.CompilerParams(needs_layout_passes=False),
        scratch_types=[
            pltpu.VMEM((rows_w, d), jnp.float32),
            pltpu.VMEM((rows_w, d), jnp.int32),
            pltpu.VMEM((rows_w, d), jnp.float32),
            pltpu.VMEM((rows_w, d), jnp.float32),
        ],
    )
    mask_sc, sel_sc = fn(x, bits)
    return mask_sc, sel_sc


def kernel(x, W1, b1, W2, b2, Wg1, bg1, Wg2, bg2):
    bsz, d = x.shape
    hdim = W1.shape[1]
    k = int(_SELECTION_RATIO * d)
    blk = min(bsz, _BLK)
    n_blocks = bsz // blk
    sc_rows = _SC_ROWS if bsz % blk == 0 and _SC_ROWS % blk == 0 \
        and _SC_ROWS % _NW == 0 and bsz - _SC_ROWS >= blk else 0
    n_sc_blocks = sc_rows // blk

    b1r = b1.reshape(1, hdim)
    b2r = b2.reshape(1, d)
    bg1r = bg1.reshape(1, hdim)
    bg2r = bg2.reshape(1, 1)

    full = lambda shape: pl.BlockSpec(shape, lambda i: (0, 0))
    rowblk = lambda shape: pl.BlockSpec(shape, lambda i: (i, 0))

    sel_tc, comb, bits, mask_tc = pl.pallas_call(
        functools.partial(_tc_body, k, n_sc_blocks),
        grid=(n_blocks,),
        in_specs=[
            rowblk((blk, d)),
            full((d, hdim)), full((1, hdim)),
            full((hdim, d)), full((1, d)),
            full((d, hdim)), full((1, hdim)),
            full((hdim, 1)), full((1, 1)),
        ],
        out_specs=[rowblk((blk, d))] * 4,
        out_shape=[jax.ShapeDtypeStruct((bsz, d), jnp.float32),
                   jax.ShapeDtypeStruct((bsz, d), jnp.float32),
                   jax.ShapeDtypeStruct((bsz, d), jnp.int32),
                   jax.ShapeDtypeStruct((bsz, d), jnp.float32)],
    )(x, W1, b1r, W2, b2r, Wg1, bg1r, Wg2, bg2r)

    if sc_rows == 0:
        return (sel_tc, comb, mask_tc)

    mask_sc, sel_sc = _sc_mask(x, bits, sc_rows, k)
    mask = lax.dynamic_update_slice(mask_tc, mask_sc, (0, 0))
    sel = lax.dynamic_update_slice(sel_tc, sel_sc, (0, 0))
    return (sel, comb, mask)


# speedup vs baseline: 1.1404x; 1.0516x over previous
import functools

import jax
import jax.numpy as jnp
from jax import lax
from jax.experimental import pallas as pl
from jax.experimental.pallas import tpu as pltpu
from jax.experimental.pallas import tpu_sc as plsc

_SELECTION_RATIO = 0.7
_N_BISECT = 30
_SC_ROWS = 1024
_NW = 32
_BLK = 1024


def _dot(a, b, precision=None):
    return jax.lax.dot_general(
        a, b, (((1,), (0,)), ((), ())),
        preferred_element_type=jnp.float32,
        precision=precision,
    )


def _tc_body(k, n_sc_blocks, x_ref, w1_ref, b1_ref, w2_ref, b2_ref, wg1_ref,
             bg1_ref, wg2_ref, bg2_ref, sel_ref, comb_ref, bits_ref,
             mask_ref):
    x = x_ref[...]
    h = jnp.maximum(_dot(x, w1_ref[...]) + b1_ref[...], 0.0)
    att = jax.nn.sigmoid(_dot(h, w2_ref[...]) + b2_ref[...])
    hg = jnp.maximum(_dot(x, wg1_ref[...]) + bg1_ref[...], 0.0)
    gate = jax.nn.sigmoid(_dot(hg, wg2_ref[...]) + bg2_ref[...])
    s = att * gate
    comb_ref[...] = s
    ki = jax.lax.bitcast_convert_type(s, jnp.int32)
    bits_ref[...] = ki

    @pl.when(pl.program_id(0) >= n_sc_blocks)
    def _selection():
        rows = s.shape[0]
        kit = ki.T
        lo_t = jnp.zeros((1, rows), jnp.int32)
        hi_t = jnp.full((1, rows), 0x40000000, jnp.int32)
        for _ in range(_N_BISECT):
            mid = (lo_t + hi_t) >> 1
            cnt = jnp.sum((kit >= mid).astype(jnp.int32), axis=0,
                          keepdims=True)
            big = cnt >= k
            lo_t = jnp.where(big, mid, lo_t)
            hi_t = jnp.where(big, hi_t, mid)
        cgt_t = jnp.sum((kit > lo_t).astype(jnp.int32), axis=0,
                        keepdims=True)
        thr = lo_t.T
        c_gt = cgt_t.T

        gt = ki > thr
        eq = ki == thr
        eq_f = eq.astype(jnp.float32)
        d = s.shape[1]
        tri = (jax.lax.broadcasted_iota(jnp.int32, (d, d), 0)
               < jax.lax.broadcasted_iota(jnp.int32, (d, d), 1)
               ).astype(jnp.float32)
        prefix = _dot(eq_f, tri)
        sel_eq = eq & (prefix < (k - c_gt).astype(jnp.float32))
        mask = (gt | sel_eq).astype(jnp.float32)

        mask_ref[...] = mask
        sel_ref[...] = x * mask


def _sc_mask_body(k, rows_w, x_hbm, bits_hbm, mask_hbm, sel_hbm,
                  xv, cv, mv, sv):
    wid = lax.axis_index("s") * 2 + lax.axis_index("c")
    base = wid * rows_w
    pltpu.sync_copy(bits_hbm.at[pl.ds(base, rows_w)], cv)
    pltpu.sync_copy(x_hbm.at[pl.ds(base, rows_w)], xv)

    kvec = jnp.full((16,), k, jnp.int32)
    one_f = jnp.full((16,), 1.0, jnp.float32)
    zero_f = jnp.zeros((16,), jnp.float32)

    def row_body(r, carry):
        kis = [cv[r, pl.ds(16 * v, 16)] for v in range(8)]
        lo = jnp.zeros((16,), jnp.int32)
        hi = jnp.full((16,), 0x40000000, jnp.int32)
        for _ in range(_N_BISECT):
            mid = (lo + hi) >> 1
            cnt = jnp.zeros((16,), jnp.int32)
            for v in range(8):
                cnt = cnt + plsc.all_reduce_population_count(kis[v] >= mid)
            big = cnt >= kvec
            lo = jnp.where(big, mid, lo)
            hi = jnp.where(big, hi, mid)
        thr = lo
        c_gt = jnp.zeros((16,), jnp.int32)
        for v in range(8):
            c_gt = c_gt + plsc.all_reduce_population_count(kis[v] > thr)
        limit = kvec - c_gt
        run = jnp.zeros((16,), jnp.int32)
        for v in range(8):
            gt = kis[v] > thr
            eq = kis[v] == thr
            eq_i = jnp.where(eq, 1, 0)
            excl = plsc.cumsum(eq_i) - eq_i + run
            sel_eq = eq & (excl < limit)
            m = jnp.where(gt | sel_eq, one_f, zero_f)
            mv[r, pl.ds(16 * v, 16)] = m
            sv[r, pl.ds(16 * v, 16)] = xv[r, pl.ds(16 * v, 16)] * m
            run = run + plsc.all_reduce_population_count(eq)
        return carry

    lax.fori_loop(0, rows_w, row_body, 0)
    pltpu.sync_copy(mv, mask_hbm.at[pl.ds(base, rows_w)])
    pltpu.sync_copy(sv, sel_hbm.at[pl.ds(base, rows_w)])


def _sc_mask(x, bits, sc_rows, k):
    d = x.shape[1]
    rows_w = sc_rows // _NW
    mesh = plsc.VectorSubcoreMesh(core_axis_name="c", subcore_axis_name="s")
    fn = pl.kernel(
        functools.partial(_sc_mask_body, k, rows_w),
        out_type=[jax.ShapeDtypeStruct((sc_rows, d), jnp.float32)] * 2,
        mesh=mesh,
        compiler_params=pltpu.CompilerParams(needs_layout_passes=False),
        scratch_types=[
            pltpu.VMEM((rows_w, d), jnp.float32),
            pltpu.VMEM((rows_w, d), jnp.int32),
            pltpu.VMEM((rows_w, d), jnp.float32),
            pltpu.VMEM((rows_w, d), jnp.float32),
        ],
    )
    mask_sc, sel_sc = fn(x, bits)
    return mask_sc, sel_sc


def kernel(x, W1, b1, W2, b2, Wg1, bg1, Wg2, bg2):
    bsz, d = x.shape
    hdim = W1.shape[1]
    k = int(_SELECTION_RATIO * d)
    blk = min(bsz, _BLK)
    n_blocks = bsz // blk
    sc_rows = _SC_ROWS if bsz % blk == 0 and _SC_ROWS % blk == 0 \
        and _SC_ROWS % _NW == 0 and bsz - _SC_ROWS >= blk else 0
    n_sc_blocks = sc_rows // blk

    b1r = b1.reshape(1, hdim)
    b2r = b2.reshape(1, d)
    bg1r = bg1.reshape(1, hdim)
    bg2r = bg2.reshape(1, 1)

    full = lambda shape: pl.BlockSpec(shape, lambda i: (0, 0))
    rowblk = lambda shape: pl.BlockSpec(shape, lambda i: (i, 0))

    sel_tc, comb, bits, mask_tc = pl.pallas_call(
        functools.partial(_tc_body, k, n_sc_blocks),
        grid=(n_blocks,),
        in_specs=[
            rowblk((blk, d)),
            full((d, hdim)), full((1, hdim)),
            full((hdim, d)), full((1, d)),
            full((d, hdim)), full((1, hdim)),
            full((hdim, 1)), full((1, 1)),
        ],
        out_specs=[rowblk((blk, d))] * 4,
        out_shape=[jax.ShapeDtypeStruct((bsz, d), jnp.float32),
                   jax.ShapeDtypeStruct((bsz, d), jnp.float32),
                   jax.ShapeDtypeStruct((bsz, d), jnp.int32),
                   jax.ShapeDtypeStruct((bsz, d), jnp.float32)],
    )(x, W1, b1r, W2, b2r, Wg1, bg1r, Wg2, bg2r)

    if sc_rows == 0:
        return (sel_tc, comb, mask_tc)

    mask_sc, sel_sc = _sc_mask(x, bits, sc_rows, k)
    mask = lax.dynamic_update_slice(mask_tc, mask_sc, (0, 0))
    sel = lax.dynamic_update_slice(sel_tc, sel_sc, (0, 0))
    return (sel, comb, mask)


# speedup vs baseline: 1.5350x; 1.3460x over previous
import functools

import jax
import jax.numpy as jnp
from jax import lax
from jax.experimental import pallas as pl
from jax.experimental.pallas import tpu as pltpu
from jax.experimental.pallas import tpu_sc as plsc

_SELECTION_RATIO = 0.7
_N_BISECT = 30
_SC_ROWS = 0
_NW = 32
_BLK = 1024


def _dot(a, b, precision=None):
    return jax.lax.dot_general(
        a, b, (((1,), (0,)), ((), ())),
        preferred_element_type=jnp.float32,
        precision=precision,
    )


def _tc_body(k, n_sc_blocks, x_ref, w1_ref, b1_ref, w2_ref, b2_ref, wg1_ref,
             bg1_ref, wg2_ref, bg2_ref, sel_ref, comb_ref, bits_ref,
             mask_ref):
    x = x_ref[...]
    h = jnp.maximum(_dot(x, w1_ref[...]) + b1_ref[...], 0.0)
    att = jax.nn.sigmoid(_dot(h, w2_ref[...]) + b2_ref[...])
    hg = jnp.maximum(_dot(x, wg1_ref[...]) + bg1_ref[...], 0.0)
    gate = jax.nn.sigmoid(_dot(hg, wg2_ref[...]) + bg2_ref[...])
    s = att * gate
    comb_ref[...] = s
    ki = jax.lax.bitcast_convert_type(s, jnp.int32)
    bits_ref[...] = ki

    @pl.when(pl.program_id(0) >= n_sc_blocks)
    def _selection():
        rows = s.shape[0]
        kit = ki.T
        lo_t = jnp.zeros((1, rows), jnp.int32)
        hi_t = jnp.full((1, rows), 0x40000000, jnp.int32)
        for _ in range(_N_BISECT):
            mid = (lo_t + hi_t) >> 1
            cnt = jnp.sum((kit >= mid).astype(jnp.int32), axis=0,
                          keepdims=True)
            big = cnt >= k
            lo_t = jnp.where(big, mid, lo_t)
            hi_t = jnp.where(big, hi_t, mid)
        cgt_t = jnp.sum((kit > lo_t).astype(jnp.int32), axis=0,
                        keepdims=True)
        thr = lo_t.T
        c_gt = cgt_t.T

        gt = ki > thr
        eq = ki == thr
        eq_f = eq.astype(jnp.float32)
        d = s.shape[1]
        tri = (jax.lax.broadcasted_iota(jnp.int32, (d, d), 0)
               < jax.lax.broadcasted_iota(jnp.int32, (d, d), 1)
               ).astype(jnp.float32)
        prefix = _dot(eq_f, tri)
        sel_eq = eq & (prefix < (k - c_gt).astype(jnp.float32))
        mask = (gt | sel_eq).astype(jnp.float32)

        mask_ref[...] = mask
        sel_ref[...] = x * mask


def _sc_mask_body(k, rows_w, x_hbm, bits_hbm, mask_hbm, sel_hbm,
                  xv, cv, mv, sv):
    wid = lax.axis_index("s") * 2 + lax.axis_index("c")
    base = wid * rows_w
    pltpu.sync_copy(bits_hbm.at[pl.ds(base, rows_w)], cv)
    pltpu.sync_copy(x_hbm.at[pl.ds(base, rows_w)], xv)

    kvec = jnp.full((16,), k, jnp.int32)
    one_f = jnp.full((16,), 1.0, jnp.float32)
    zero_f = jnp.zeros((16,), jnp.float32)

    def row_body(r, carry):
        kis = [cv[r, pl.ds(16 * v, 16)] for v in range(8)]
        lo = jnp.zeros((16,), jnp.int32)
        hi = jnp.full((16,), 0x40000000, jnp.int32)
        for _ in range(_N_BISECT):
            mid = (lo + hi) >> 1
            cnt = jnp.zeros((16,), jnp.int32)
            for v in range(8):
                cnt = cnt + plsc.all_reduce_population_count(kis[v] >= mid)
            big = cnt >= kvec
            lo = jnp.where(big, mid, lo)
            hi = jnp.where(big, hi, mid)
        thr = lo
        c_gt = jnp.zeros((16,), jnp.int32)
        for v in range(8):
            c_gt = c_gt + plsc.all_reduce_population_count(kis[v] > thr)
        limit = kvec - c_gt
        run = jnp.zeros((16,), jnp.int32)
        for v in range(8):
            gt = kis[v] > thr
            eq = kis[v] == thr
            eq_i = jnp.where(eq, 1, 0)
            excl = plsc.cumsum(eq_i) - eq_i + run
            sel_eq = eq & (excl < limit)
            m = jnp.where(gt | sel_eq, one_f, zero_f)
            mv[r, pl.ds(16 * v, 16)] = m
            sv[r, pl.ds(16 * v, 16)] = xv[r, pl.ds(16 * v, 16)] * m
            run = run + plsc.all_reduce_population_count(eq)
        return carry

    lax.fori_loop(0, rows_w, row_body, 0)
    pltpu.sync_copy(mv, mask_hbm.at[pl.ds(base, rows_w)])
    pltpu.sync_copy(sv, sel_hbm.at[pl.ds(base, rows_w)])


def _sc_mask(x, bits, sc_rows, k):
    d = x.shape[1]
    rows_w = sc_rows // _NW
    mesh = plsc.VectorSubcoreMesh(core_axis_name="c", subcore_axis_name="s")
    fn = pl.kernel(
        functools.partial(_sc_mask_body, k, rows_w),
        out_type=[jax.ShapeDtypeStruct((sc_rows, d), jnp.float32)] * 2,
        mesh=mesh,
        compiler_params=pltpu.CompilerParams(needs_layout_passes=False),
        scratch_types=[
            pltpu.VMEM((rows_w, d), jnp.float32),
            pltpu.VMEM((rows_w, d), jnp.int32),
            pltpu.VMEM((rows_w, d), jnp.float32),
            pltpu.VMEM((rows_w, d), jnp.float32),
        ],
    )
    mask_sc, sel_sc = fn(x, bits)
    return mask_sc, sel_sc


def kernel(x, W1, b1, W2, b2, Wg1, bg1, Wg2, bg2):
    bsz, d = x.shape
    hdim = W1.shape[1]
    k = int(_SELECTION_RATIO * d)
    blk = min(bsz, _BLK)
    n_blocks = bsz // blk
    sc_rows = _SC_ROWS if bsz % blk == 0 and _SC_ROWS % blk == 0 \
        and _SC_ROWS % _NW == 0 and bsz - _SC_ROWS >= blk else 0
    n_sc_blocks = sc_rows // blk

    b1r = b1.reshape(1, hdim)
    b2r = b2.reshape(1, d)
    bg1r = bg1.reshape(1, hdim)
    bg2r = bg2.reshape(1, 1)

    full = lambda shape: pl.BlockSpec(shape, lambda i: (0, 0))
    rowblk = lambda shape: pl.BlockSpec(shape, lambda i: (i, 0))

    sel_tc, comb, bits, mask_tc = pl.pallas_call(
        functools.partial(_tc_body, k, n_sc_blocks),
        grid=(n_blocks,),
        in_specs=[
            rowblk((blk, d)),
            full((d, hdim)), full((1, hdim)),
            full((hdim, d)), full((1, d)),
            full((d, hdim)), full((1, hdim)),
            full((hdim, 1)), full((1, 1)),
        ],
        out_specs=[rowblk((blk, d))] * 4,
        out_shape=[jax.ShapeDtypeStruct((bsz, d), jnp.float32),
                   jax.ShapeDtypeStruct((bsz, d), jnp.float32),
                   jax.ShapeDtypeStruct((bsz, d), jnp.int32),
                   jax.ShapeDtypeStruct((bsz, d), jnp.float32)],
    )(x, W1, b1r, W2, b2r, Wg1, bg1r, Wg2, bg2r)

    if sc_rows == 0:
        return (sel_tc, comb, mask_tc)

    mask_sc, sel_sc = _sc_mask(x, bits, sc_rows, k)
    mask = lax.dynamic_update_slice(mask_tc, mask_sc, (0, 0))
    sel = lax.dynamic_update_slice(sel_tc, sel_sc, (0, 0))
    return (sel, comb, mask)
